# Initial kernel scaffold; baseline (speedup 1.0000x reference)
import jax
import jax.numpy as jnp
from jax.experimental import pallas as pl


def kernel(month_idx, day_idx, sp_idx, dtype_idx, emb_month, emb_day, emb_sp, emb_dtype):
    raise NotImplementedError("write your pallas kernel here")


# speedup vs baseline: 14.5910x; 14.5910x over previous
import functools

import jax
import jax.numpy as jnp
from jax import lax
from jax.experimental import pallas as pl
from jax.experimental.pallas import tpu as pltpu
from jax.experimental.pallas import tpu_sc as plsc

NC, NS, L = 2, 16, 16
NW = NC * NS
B, T = 16384, 200
N = B * T
OUT_D = 15
PER_W = N // NW
CHUNK = 2048
N_CHUNKS = PER_W // CHUNK

MB, DB, SB, TB = 0, 48, 69, 369
TAB_PAD = 384

_mesh = plsc.VectorSubcoreMesh(core_axis_name="c", subcore_axis_name="s")


@functools.partial(
    pl.kernel,
    out_type=jax.ShapeDtypeStruct((N * OUT_D,), jnp.float32),
    mesh=_mesh,
    compiler_params=pltpu.CompilerParams(needs_layout_passes=False),
    scratch_types=[
        pltpu.VMEM((TAB_PAD,), jnp.float32),
        pltpu.VMEM((CHUNK,), jnp.int32),
        pltpu.VMEM((CHUNK,), jnp.int32),
        pltpu.VMEM((CHUNK,), jnp.int32),
        pltpu.VMEM((CHUNK,), jnp.int32),
        pltpu.VMEM((CHUNK * OUT_D,), jnp.float32),
    ],
)
def _emb_lookup(mi, di, si, ti, tab, out_hbm, tab_v, mi_v, di_v, si_v, ti_v,
                out_v):
    wid = lax.axis_index("s") * NC + lax.axis_index("c")
    base = wid * PER_W
    pltpu.sync_copy(tab, tab_v)
    obase0 = lax.iota(jnp.int32, L) * OUT_D

    def chunk_body(k, carry):
        cbase = base + k * CHUNK
        pltpu.sync_copy(mi.at[pl.ds(cbase, CHUNK)], mi_v)
        pltpu.sync_copy(di.at[pl.ds(cbase, CHUNK)], di_v)
        pltpu.sync_copy(si.at[pl.ds(cbase, CHUNK)], si_v)
        pltpu.sync_copy(ti.at[pl.ds(cbase, CHUNK)], ti_v)

        def inner(i, c):
            m = mi_v[pl.ds(i * L, L)]
            d = di_v[pl.ds(i * L, L)]
            s = si_v[pl.ds(i * L, L)]
            t = ti_v[pl.ds(i * L, L)]
            ob = obase0 + i * (L * OUT_D)
            addr = [m * 4 + (MB + j) for j in range(4)]
            addr += [d * 3 + (DB + j) for j in range(3)]
            addr += [s * 6 + (SB + j) for j in range(6)]
            addr += [t * 2 + (TB + j) for j in range(2)]
            for col, a in enumerate(addr):
                plsc.store_scatter(out_v, [ob + col],
                                   plsc.load_gather(tab_v, [a]))
            return c

        lax.fori_loop(0, CHUNK // L, inner, 0, unroll=False)
        pltpu.sync_copy(out_v,
                        out_hbm.at[pl.ds(cbase * OUT_D, CHUNK * OUT_D)])
        return carry

    lax.fori_loop(0, N_CHUNKS, chunk_body, 0, unroll=False)


def kernel(month_idx, day_idx, sp_idx, dtype_idx, emb_month, emb_day, emb_sp,
           emb_dtype):
    mi = month_idx.astype(jnp.int32).reshape(N)
    di = day_idx.astype(jnp.int32).reshape(N)
    si = sp_idx.astype(jnp.int32).reshape(N)
    ti = dtype_idx.astype(jnp.int32).reshape(N)
    tab = jnp.concatenate([
        emb_month.reshape(-1),
        emb_day.reshape(-1),
        emb_sp.reshape(-1),
        emb_dtype.reshape(-1),
        jnp.zeros((TAB_PAD - 373,), jnp.float32),
    ])
    out = _emb_lookup(mi, di, si, ti, tab)
    return out.reshape(B, T, OUT_D)


# speedup vs baseline: 16.4407x; 1.1268x over previous
import functools

import jax
import jax.numpy as jnp
from jax import lax
from jax.experimental import pallas as pl
from jax.experimental.pallas import tpu as pltpu
from jax.experimental.pallas import tpu_sc as plsc

NC, NS, L = 2, 16, 16
NW = NC * NS
B, T = 16384, 200
N = B * T
OUT_D = 15
PER_W = N // NW
CHUNK = 2048
N_CHUNKS = PER_W // CHUNK

MB, DB, SB, TB = 0, 48, 69, 369
TAB_PAD = 384

_mesh = plsc.VectorSubcoreMesh(core_axis_name="c", subcore_axis_name="s")


@functools.partial(
    pl.kernel,
    out_type=jax.ShapeDtypeStruct((N * OUT_D,), jnp.float32),
    mesh=_mesh,
    compiler_params=pltpu.CompilerParams(needs_layout_passes=False),
    scratch_types=[
        pltpu.VMEM((TAB_PAD,), jnp.float32),
        pltpu.VMEM((CHUNK,), jnp.int32),
        pltpu.VMEM((CHUNK,), jnp.int32),
        pltpu.VMEM((CHUNK,), jnp.int32),
        pltpu.VMEM((CHUNK,), jnp.int32),
        pltpu.VMEM((CHUNK * OUT_D,), jnp.float32),
    ],
)
def _emb_lookup(mi, di, si, ti, tab, out_hbm, tab_v, mi_v, di_v, si_v, ti_v,
                out_v):
    wid = lax.axis_index("s") * NC + lax.axis_index("c")
    base = wid * PER_W
    pltpu.sync_copy(tab, tab_v)
    obase0 = lax.iota(jnp.int32, L) * OUT_D

    def chunk_body(k, carry):
        cbase = base + k * CHUNK
        pltpu.sync_copy(mi.at[pl.ds(cbase, CHUNK)], mi_v)
        pltpu.sync_copy(di.at[pl.ds(cbase, CHUNK)], di_v)
        pltpu.sync_copy(si.at[pl.ds(cbase, CHUNK)], si_v)
        pltpu.sync_copy(ti.at[pl.ds(cbase, CHUNK)], ti_v)

        @plsc.parallel_loop(0, CHUNK, step=L, unroll=4)
        def inner(i):
            m = mi_v[pl.ds(i, L)]
            d = di_v[pl.ds(i, L)]
            s = si_v[pl.ds(i, L)]
            t = ti_v[pl.ds(i, L)]
            ob = obase0 + i * OUT_D
            addr = [m * 4 + (MB + j) for j in range(4)]
            addr += [d * 3 + (DB + j) for j in range(3)]
            addr += [s * 6 + (SB + j) for j in range(6)]
            addr += [t * 2 + (TB + j) for j in range(2)]
            for col, a in enumerate(addr):
                plsc.store_scatter(out_v, [ob + col],
                                   plsc.load_gather(tab_v, [a]))
        pltpu.sync_copy(out_v,
                        out_hbm.at[pl.ds(cbase * OUT_D, CHUNK * OUT_D)])
        return carry

    lax.fori_loop(0, N_CHUNKS, chunk_body, 0, unroll=False)


def kernel(month_idx, day_idx, sp_idx, dtype_idx, emb_month, emb_day, emb_sp,
           emb_dtype):
    mi = month_idx.astype(jnp.int32).reshape(N)
    di = day_idx.astype(jnp.int32).reshape(N)
    si = sp_idx.astype(jnp.int32).reshape(N)
    ti = dtype_idx.astype(jnp.int32).reshape(N)
    tab = jnp.concatenate([
        emb_month.reshape(-1),
        emb_day.reshape(-1),
        emb_sp.reshape(-1),
        emb_dtype.reshape(-1),
        jnp.zeros((TAB_PAD - 373,), jnp.float32),
    ])
    out = _emb_lookup(mi, di, si, ti, tab)
    return out.reshape(B, T, OUT_D)


# speedup vs baseline: 19.8106x; 1.2050x over previous
import functools

import jax
import jax.numpy as jnp
from jax import lax
from jax.experimental import pallas as pl
from jax.experimental.pallas import tpu as pltpu
from jax.experimental.pallas import tpu_sc as plsc

NC, NS, L = 2, 16, 16
NW = NC * NS
B, T = 16384, 200
N = B * T
OUT_D = 15
ROWS_W = B // NW
RB = 16
OB = 4
N_CHUNKS = ROWS_W // RB
NFULL = T // L
TAILC = T - NFULL * L

MB, DB, SB, TB = 0, 48, 69, 369
TAB_PAD = 384

_mesh = plsc.VectorSubcoreMesh(core_axis_name="c", subcore_axis_name="s")


@functools.partial(
    pl.kernel,
    out_type=jax.ShapeDtypeStruct((N, OUT_D), jnp.float32),
    mesh=_mesh,
    compiler_params=pltpu.CompilerParams(needs_layout_passes=False),
    scratch_types=[
        pltpu.VMEM((TAB_PAD,), jnp.float32),
        pltpu.VMEM((RB, T), jnp.int32),
        pltpu.VMEM((RB, T), jnp.int32),
        pltpu.VMEM((RB, T), jnp.int32),
        pltpu.VMEM((RB, T), jnp.int32),
        pltpu.VMEM((OB * T, OUT_D), jnp.float32),
    ],
)
def _emb_lookup(mi, di, si, ti, tab, out_hbm, tab_v, mi_v, di_v, si_v, ti_v,
                out_v):
    wid = lax.axis_index("s") * NC + lax.axis_index("c")
    row0 = wid * ROWS_W
    pltpu.sync_copy(tab, tab_v)
    lanes = lax.iota(jnp.int32, L)
    tail_row = lax.select(lanes >= TAILC, jnp.full((L,), 1, jnp.int32),
                          jnp.full((L,), 0, jnp.int32))
    tail_col = (lanes & (TAILC - 1)) + NFULL * L

    def gather_group(m, d, s, t, ebase):
        addr = [m * 4 + (MB + j) for j in range(4)]
        addr += [d * 3 + (DB + j) for j in range(3)]
        addr += [s * 6 + (SB + j) for j in range(6)]
        addr += [t * 2 + (TB + j) for j in range(2)]
        for col, a in enumerate(addr):
            plsc.store_scatter(out_v, [ebase, jnp.full((L,), col, jnp.int32)],
                               plsc.load_gather(tab_v, [a]))

    def chunk_body(k, carry):
        rbase = row0 + k * RB
        pltpu.sync_copy(mi.at[pl.ds(rbase, RB), :], mi_v)
        pltpu.sync_copy(di.at[pl.ds(rbase, RB), :], di_v)
        pltpu.sync_copy(si.at[pl.ds(rbase, RB), :], si_v)
        pltpu.sync_copy(ti.at[pl.ds(rbase, RB), :], ti_v)

        def out_block(ob, _):
            @plsc.parallel_loop(0, OB, step=2)
            def row_pair(rl):
                r = ob * OB + rl
                for rr in range(2):
                    row = r + rr
                    orow = rl + rr
                    for cg in range(NFULL):
                        m = mi_v[row, pl.ds(cg * L, L)]
                        d = di_v[row, pl.ds(cg * L, L)]
                        s = si_v[row, pl.ds(cg * L, L)]
                        t = ti_v[row, pl.ds(cg * L, L)]
                        ebase = (orow * T + cg * L) + lanes
                        gather_group(m, d, s, t, ebase)
                trow = r + tail_row
                m = plsc.load_gather(mi_v, [trow, tail_col])
                d = plsc.load_gather(di_v, [trow, tail_col])
                s = plsc.load_gather(si_v, [trow, tail_col])
                t = plsc.load_gather(ti_v, [trow, tail_col])
                ebase = (rl + tail_row) * T + tail_col
                gather_group(m, d, s, t, ebase)

            pltpu.sync_copy(
                out_v,
                out_hbm.at[pl.ds((rbase + ob * OB) * T, OB * T), :])
            return _

        lax.fori_loop(0, RB // OB, out_block, 0, unroll=False)
        return carry

    lax.fori_loop(0, N_CHUNKS, chunk_body, 0, unroll=False)


def kernel(month_idx, day_idx, sp_idx, dtype_idx, emb_month, emb_day, emb_sp,
           emb_dtype):
    mi = month_idx.astype(jnp.int32)
    di = day_idx.astype(jnp.int32)
    si = sp_idx.astype(jnp.int32)
    ti = dtype_idx.astype(jnp.int32)
    tab = jnp.concatenate([
        emb_month.reshape(-1),
        emb_day.reshape(-1),
        emb_sp.reshape(-1),
        emb_dtype.reshape(-1),
        jnp.zeros((TAB_PAD - 373,), jnp.float32),
    ])
    out = _emb_lookup(mi, di, si, ti, tab)
    return out.reshape(B, T, OUT_D)


# speedup vs baseline: 110.5581x; 5.5808x over previous
import functools

import jax
import jax.numpy as jnp
from jax import lax
from jax.experimental import pallas as pl
from jax.experimental.pallas import tpu as pltpu
from jax.experimental.pallas import tpu_sc as plsc

NC, NS, L = 2, 16, 16
NW = NC * NS
B, T = 16384, 200
OUT_D = 15
BW = B // NW
TR = 8
NT = T // TR
NG = BW // L

MB, DB, SB, TB = 0, 48, 69, 369
TAB_PAD = 384

_mesh = plsc.VectorSubcoreMesh(core_axis_name="c", subcore_axis_name="s")


@functools.partial(
    pl.kernel,
    out_type=jax.ShapeDtypeStruct((OUT_D, T, B), jnp.float32),
    mesh=_mesh,
    compiler_params=pltpu.CompilerParams(needs_layout_passes=False),
    scratch_types=[
        pltpu.VMEM((TAB_PAD,), jnp.float32),
        pltpu.VMEM((TR, BW), jnp.int32),
        pltpu.VMEM((TR, BW), jnp.int32),
        pltpu.VMEM((TR, BW), jnp.int32),
        pltpu.VMEM((TR, BW), jnp.int32),
        pltpu.VMEM((OUT_D, TR, BW), jnp.float32),
    ],
)
def _emb_lookup(mi, di, si, ti, tab, out_hbm, tab_v, mi_v, di_v, si_v, ti_v,
                out_v):
    wid = lax.axis_index("s") * NC + lax.axis_index("c")
    b0 = wid * BW
    pltpu.sync_copy(tab, tab_v)

    def t_step(tt, carry):
        r0 = tt * TR
        pltpu.sync_copy(mi.at[pl.ds(r0, TR), pl.ds(b0, BW)], mi_v)
        pltpu.sync_copy(di.at[pl.ds(r0, TR), pl.ds(b0, BW)], di_v)
        pltpu.sync_copy(si.at[pl.ds(r0, TR), pl.ds(b0, BW)], si_v)
        pltpu.sync_copy(ti.at[pl.ds(r0, TR), pl.ds(b0, BW)], ti_v)

        @plsc.parallel_loop(0, BW, step=L, unroll=2)
        def group(g):
            for r in range(TR):
                m = mi_v[r, pl.ds(g, L)]
                d = di_v[r, pl.ds(g, L)]
                s = si_v[r, pl.ds(g, L)]
                t = ti_v[r, pl.ds(g, L)]
                addr = [m * 4 + (MB + j) for j in range(4)]
                addr += [d * 3 + (DB + j) for j in range(3)]
                addr += [s * 6 + (SB + j) for j in range(6)]
                addr += [t * 2 + (TB + j) for j in range(2)]
                for f, a in enumerate(addr):
                    out_v[f, r, pl.ds(g, L)] = plsc.load_gather(tab_v, [a])

        pltpu.sync_copy(out_v,
                        out_hbm.at[:, pl.ds(r0, TR), pl.ds(b0, BW)])
        return carry

    lax.fori_loop(0, NT, t_step, 0, unroll=False)


def kernel(month_idx, day_idx, sp_idx, dtype_idx, emb_month, emb_day, emb_sp,
           emb_dtype):
    mi = month_idx.astype(jnp.int32).T
    di = day_idx.astype(jnp.int32).T
    si = sp_idx.astype(jnp.int32).T
    ti = dtype_idx.astype(jnp.int32).T
    tab = jnp.concatenate([
        emb_month.reshape(-1),
        emb_day.reshape(-1),
        emb_sp.reshape(-1),
        emb_dtype.reshape(-1),
        jnp.zeros((TAB_PAD - 373,), jnp.float32),
    ])
    out = _emb_lookup(mi, di, si, ti, tab)
    return out.transpose(2, 1, 0)


# speedup vs baseline: 154.3095x; 1.3957x over previous
import functools

import jax
import jax.numpy as jnp
from jax import lax
from jax.experimental import pallas as pl
from jax.experimental.pallas import tpu as pltpu
from jax.experimental.pallas import tpu_sc as plsc

NC, NS, L = 2, 16, 16
NW = NC * NS
B, T = 16384, 200
OUT_D = 15
BW = B // NW
HB = 256
TR = 8
NT = T // TR

MB, DB, SB, TB = 0, 48, 69, 369
TAB_PAD = 384

_mesh = plsc.VectorSubcoreMesh(core_axis_name="c", subcore_axis_name="s")

_IDX_BUF = pltpu.VMEM((TR, HB), jnp.int32)
_OUT_BUF = pltpu.VMEM((OUT_D, TR, HB), jnp.float32)


@functools.partial(
    pl.kernel,
    out_type=jax.ShapeDtypeStruct((OUT_D, T, B), jnp.float32),
    mesh=_mesh,
    compiler_params=pltpu.CompilerParams(needs_layout_passes=False),
    scratch_types=[
        pltpu.VMEM((TAB_PAD,), jnp.float32),
        _IDX_BUF, _IDX_BUF, _IDX_BUF, _IDX_BUF,
        _IDX_BUF, _IDX_BUF, _IDX_BUF, _IDX_BUF,
        _OUT_BUF, _OUT_BUF,
        pltpu.SemaphoreType.DMA, pltpu.SemaphoreType.DMA,
        pltpu.SemaphoreType.DMA, pltpu.SemaphoreType.DMA,
    ],
)
def _emb_lookup(mi, di, si, ti, tab, out_hbm, tab_v,
                mi_a, di_a, si_a, ti_a, mi_b, di_b, si_b, ti_b,
                out_a, out_b, isem_a, isem_b, osem_a, osem_b):
    wid = lax.axis_index("s") * NC + lax.axis_index("c")
    b0 = wid * BW
    pltpu.sync_copy(tab, tab_v)

    bufs = ((mi_a, di_a, si_a, ti_a, out_a, isem_a, osem_a),
            (mi_b, di_b, si_b, ti_b, out_b, isem_b, osem_b))

    def in_slices(tt, h):
        r0 = tt * TR
        bh = b0 + h * HB
        return [src.at[pl.ds(r0, TR), pl.ds(bh, HB)]
                for src in (mi, di, si, ti)]

    def start_in(tt, h):
        bm, bd, bs, bt, _, isem, _ = bufs[h]
        for src, dst in zip(in_slices(tt, h), (bm, bd, bs, bt)):
            pltpu.async_copy(src, dst, isem)

    def wait_in(tt, h):
        bm, bd, bs, bt, _, isem, _ = bufs[h]
        for src, dst in zip(in_slices(tt, h), (bm, bd, bs, bt)):
            pltpu.make_async_copy(src, dst, isem).wait()

    def out_slice(tt, h):
        return out_hbm.at[:, pl.ds(tt * TR, TR), pl.ds(b0 + h * HB, HB)]

    def start_out(tt, h):
        ov, osem = bufs[h][4], bufs[h][6]
        pltpu.async_copy(ov, out_slice(tt, h), osem)

    def wait_out(tt, h):
        ov, osem = bufs[h][4], bufs[h][6]
        pltpu.make_async_copy(ov, out_slice(tt, h), osem).wait()

    def compute(h):
        bm, bd, bs, bt, ov = bufs[h][:5]

        @plsc.parallel_loop(0, HB, step=L, unroll=2)
        def group(g):
            for r in range(TR):
                m = bm[r, pl.ds(g, L)]
                d = bd[r, pl.ds(g, L)]
                s = bs[r, pl.ds(g, L)]
                t = bt[r, pl.ds(g, L)]
                addr = [m * 4 + (MB + j) for j in range(4)]
                addr += [d * 3 + (DB + j) for j in range(3)]
                addr += [s * 6 + (SB + j) for j in range(6)]
                addr += [t * 2 + (TB + j) for j in range(2)]
                for f, a in enumerate(addr):
                    ov[f, r, pl.ds(g, L)] = plsc.load_gather(tab_v, [a])

    start_in(0, 0)

    def t_step(tt, carry):
        start_in(tt, 1)
        wait_in(tt, 0)

        @pl.when(tt > 0)
        def _():
            wait_out(tt, 0)

        compute(0)
        start_out(tt, 0)

        @pl.when(tt + 1 < NT)
        def _():
            start_in(tt + 1, 0)

        wait_in(tt, 1)

        @pl.when(tt > 0)
        def _():
            wait_out(tt, 1)

        compute(1)
        start_out(tt, 1)
        return carry

    lax.fori_loop(0, NT, t_step, 0, unroll=False)
    wait_out(NT - 1, 0)
    wait_out(NT - 1, 1)


def kernel(month_idx, day_idx, sp_idx, dtype_idx, emb_month, emb_day, emb_sp,
           emb_dtype):
    mi = month_idx.astype(jnp.int32).T
    di = day_idx.astype(jnp.int32).T
    si = sp_idx.astype(jnp.int32).T
    ti = dtype_idx.astype(jnp.int32).T
    tab = jnp.concatenate([
        emb_month.reshape(-1),
        emb_day.reshape(-1),
        emb_sp.reshape(-1),
        emb_dtype.reshape(-1),
        jnp.zeros((TAB_PAD - 373,), jnp.float32),
    ])
    out = _emb_lookup(mi, di, si, ti, tab)
    return out.transpose(2, 1, 0)
